# Initial kernel scaffold; baseline (speedup 1.0000x reference)
import jax
import jax.numpy as jnp
from jax.experimental import pallas as pl


def kernel(x, emb, fc_w, fc_b):
    raise NotImplementedError("write your pallas kernel here")


# speedup vs baseline: 8.2983x; 8.2983x over previous
import functools

import jax
import jax.numpy as jnp
from jax import lax
from jax.experimental import pallas as pl
from jax.experimental.pallas import tpu as pltpu
from jax.experimental.pallas import tpu_sc as plsc

VOCAB = 1000000
NUM_CLASSES = 100
EMB_DIM = 32
B = 16384
L = 200
HALF_L = L // 2

NUM_WORKERS = 32
ROWS_PER_WORKER = B // NUM_WORKERS


def _sc_body(x_hbm, emb_hbm, out_hbm, idx_v, g0, g1, sums_v, sem):
    nc = 2
    wid = lax.axis_index("s") * nc + lax.axis_index("c")
    row_base = wid * ROWS_PER_WORKER

    def row_work(r, _):
        pltpu.sync_copy(x_hbm.at[pl.ds(2 * (row_base + r), 2)], idx_v)
        cp0 = pltpu.async_copy(emb_hbm.at[idx_v.at[0]], g0, sem)
        cp1 = pltpu.async_copy(emb_hbm.at[idx_v.at[1]], g1, sem)
        cp0.wait()
        cp1.wait()

        zero = jnp.zeros((16,), jnp.float32)

        def red(l, accs):
            a0, a1 = accs
            a0 = a0 + g0[l, pl.ds(0, 16)] + g1[l, pl.ds(0, 16)]
            a1 = a1 + g0[l, pl.ds(16, 16)] + g1[l, pl.ds(16, 16)]
            return (a0, a1)

        a0, a1 = lax.fori_loop(0, HALF_L, red, (zero, zero))
        sums_v[r, pl.ds(0, 16)] = a0
        sums_v[r, pl.ds(16, 16)] = a1
        return 0

    lax.fori_loop(0, ROWS_PER_WORKER, row_work, 0)
    pltpu.sync_copy(sums_v, out_hbm.at[pl.ds(row_base, ROWS_PER_WORKER)])


def _sc_pooled_sums(x_r, emb):
    mesh = plsc.VectorSubcoreMesh(core_axis_name="c", subcore_axis_name="s")
    return pl.kernel(
        _sc_body,
        out_type=jax.ShapeDtypeStruct((B, EMB_DIM), jnp.float32),
        mesh=mesh,
        compiler_params=pltpu.CompilerParams(use_tc_tiling_on_sc=False),
        scratch_types=[
            pltpu.VMEM((2, HALF_L), jnp.int32),
            pltpu.VMEM((HALF_L, EMB_DIM), jnp.float32),
            pltpu.VMEM((HALF_L, EMB_DIM), jnp.float32),
            pltpu.VMEM((ROWS_PER_WORKER, EMB_DIM), jnp.float32),
            pltpu.SemaphoreType.DMA,
        ],
    )(x_r, emb)


def _tc_body(x_ref, sums_ref, w_ref, b_ref, out_ref):
    cnt = jnp.sum((x_ref[...] != 0).astype(jnp.float32), axis=1, keepdims=True)
    denom = jnp.maximum(cnt, 1.0)
    avg = sums_ref[...] / denom
    out_ref[...] = (
        lax.dot_general(avg, w_ref[...], (((1,), (1,)), ((), ())),
                        preferred_element_type=jnp.float32)
        + b_ref[...]
    )


def _tc_logits(x, sums, fc_w, fc_b):
    blk = 2048
    grid = (B // blk,)
    return pl.pallas_call(
        _tc_body,
        grid=grid,
        in_specs=[
            pl.BlockSpec((blk, L), lambda i: (i, 0)),
            pl.BlockSpec((blk, EMB_DIM), lambda i: (i, 0)),
            pl.BlockSpec((NUM_CLASSES, EMB_DIM), lambda i: (0, 0)),
            pl.BlockSpec((1, NUM_CLASSES), lambda i: (0, 0)),
        ],
        out_specs=pl.BlockSpec((blk, NUM_CLASSES), lambda i: (i, 0)),
        out_shape=jax.ShapeDtypeStruct((B, NUM_CLASSES), jnp.float32),
    )(x, sums, fc_w, fc_b)


def kernel(x, emb, fc_w, fc_b):
    x = x.astype(jnp.int32)
    x_r = x.reshape(2 * B, HALF_L)
    sums = _sc_pooled_sums(x_r, emb)
    return _tc_logits(x, sums, fc_w, fc_b.reshape(1, NUM_CLASSES))


# speedup vs baseline: 16.1472x; 1.9458x over previous
import functools

import jax
import jax.numpy as jnp
from jax import lax
from jax.experimental import pallas as pl
from jax.experimental.pallas import tpu as pltpu
from jax.experimental.pallas import tpu_sc as plsc

VOCAB = 1000000
NUM_CLASSES = 100
EMB_DIM = 32
B = 16384
L = 200
HALF_L = L // 2

NUM_WORKERS = 32
ROWS_PER_WORKER = B // NUM_WORKERS


G = 8
NG = ROWS_PER_WORKER // G
GROW = G * L
UNROLL = 8


def _sc_body(x_hbm, emb_hbm, out_hbm, idxb, gbuf, sums_v,
             gsem0, gsem1, isem0, isem1):
    nc = 2
    wid = lax.axis_index("s") * nc + lax.axis_index("c")
    row_base = wid * ROWS_PER_WORKER
    xrow_base = 2 * row_base

    def idx_src(g):
        return x_hbm.at[pl.ds(xrow_base + g * 2 * G, 2 * G)]

    def fire(par, gsem):
        for r in range(G):
            for h in range(2):
                pltpu.async_copy(
                    emb_hbm.at[idxb.at[par, 2 * r + h]],
                    gbuf.at[par, pl.ds(r * L + h * HALF_L, HALF_L)],
                    gsem,
                )

    def drain(par, gsem):
        pltpu.make_async_copy(
            emb_hbm.at[pl.ds(0, GROW)], gbuf.at[par], gsem
        ).wait()

    def reduce_group(g, par):
        zero = jnp.zeros((16,), jnp.float32)
        for r in range(G):
            ro = r * L

            def red(l2, accs, _ro=ro):
                a0, a1 = accs
                for u in range(UNROLL):
                    off = _ro + l2 * UNROLL + u
                    a0 = a0 + gbuf[par, off, pl.ds(0, 16)]
                    a1 = a1 + gbuf[par, off, pl.ds(16, 16)]
                return (a0, a1)

            a0, a1 = lax.fori_loop(0, L // UNROLL, red, (zero, zero))
            out_row = g * G + r
            sums_v[out_row, pl.ds(0, 16)] = a0
            sums_v[out_row, pl.ds(16, 16)] = a1

    def group_iter(g, par, gsem_cur, gsem_next, isem_next, isem_cur):
        drain(par, gsem_cur)

        @pl.when(g + 1 < NG)
        def _():
            pltpu.make_async_copy(idx_src(g + 1), idxb.at[1 - par],
                                  isem_next).wait()
            fire(1 - par, gsem_next)

        @pl.when(g + 2 < NG)
        def _():
            pltpu.async_copy(idx_src(g + 2), idxb.at[par], isem_cur)

        reduce_group(g, par)

    pltpu.sync_copy(idx_src(0), idxb.at[0])
    fire(0, gsem0)
    pltpu.async_copy(idx_src(1), idxb.at[1], isem1)

    def two_groups(gp, _):
        g = 2 * gp
        group_iter(g, 0, gsem0, gsem1, isem1, isem0)
        group_iter(g + 1, 1, gsem1, gsem0, isem0, isem1)
        return 0

    lax.fori_loop(0, NG // 2, two_groups, 0)
    pltpu.sync_copy(sums_v, out_hbm.at[pl.ds(row_base, ROWS_PER_WORKER)])


def _sc_pooled_sums(x_r, emb):
    mesh = plsc.VectorSubcoreMesh(core_axis_name="c", subcore_axis_name="s")
    return pl.kernel(
        _sc_body,
        out_type=jax.ShapeDtypeStruct((B, EMB_DIM), jnp.float32),
        mesh=mesh,
        compiler_params=pltpu.CompilerParams(use_tc_tiling_on_sc=False),
        scratch_types=[
            pltpu.VMEM((2, 2 * G, HALF_L), jnp.int32),
            pltpu.VMEM((2, GROW, EMB_DIM), jnp.float32),
            pltpu.VMEM((ROWS_PER_WORKER, EMB_DIM), jnp.float32),
            pltpu.SemaphoreType.DMA,
            pltpu.SemaphoreType.DMA,
            pltpu.SemaphoreType.DMA,
            pltpu.SemaphoreType.DMA,
        ],
    )(x_r, emb)


def _tc_body(x_ref, sums_ref, w_ref, b_ref, out_ref):
    cnt = jnp.sum((x_ref[...] != 0).astype(jnp.float32), axis=1, keepdims=True)
    denom = jnp.maximum(cnt, 1.0)
    avg = sums_ref[...] / denom
    out_ref[...] = (
        lax.dot_general(avg, w_ref[...], (((1,), (1,)), ((), ())),
                        preferred_element_type=jnp.float32)
        + b_ref[...]
    )


def _tc_logits(x, sums, fc_w, fc_b):
    blk = 2048
    grid = (B // blk,)
    return pl.pallas_call(
        _tc_body,
        grid=grid,
        in_specs=[
            pl.BlockSpec((blk, L), lambda i: (i, 0)),
            pl.BlockSpec((blk, EMB_DIM), lambda i: (i, 0)),
            pl.BlockSpec((NUM_CLASSES, EMB_DIM), lambda i: (0, 0)),
            pl.BlockSpec((1, NUM_CLASSES), lambda i: (0, 0)),
        ],
        out_specs=pl.BlockSpec((blk, NUM_CLASSES), lambda i: (i, 0)),
        out_shape=jax.ShapeDtypeStruct((B, NUM_CLASSES), jnp.float32),
    )(x, sums, fc_w, fc_b)


def kernel(x, emb, fc_w, fc_b):
    x = x.astype(jnp.int32)
    x_r = x.reshape(2 * B, HALF_L)
    sums = _sc_pooled_sums(x_r, emb)
    return _tc_logits(x, sums, fc_w, fc_b.reshape(1, NUM_CLASSES))


# speedup vs baseline: 17.5827x; 1.0889x over previous
import functools

import jax
import jax.numpy as jnp
from jax import lax
from jax.experimental import pallas as pl
from jax.experimental.pallas import tpu as pltpu
from jax.experimental.pallas import tpu_sc as plsc

VOCAB = 1000000
NUM_CLASSES = 100
EMB_DIM = 32
B = 16384
L = 200
HALF_L = L // 2

NUM_WORKERS = 32
ROWS_PER_WORKER = B // NUM_WORKERS


G = 8
NG = ROWS_PER_WORKER // G
GROW = G * L
UNROLL = 8


def _sc_body(x_hbm, emb_hbm, out_hbm, idxb, gbuf, sums_v,
             gsem0, gsem1, isem0, isem1):
    nc = 2
    wid = lax.axis_index("s") * nc + lax.axis_index("c")
    row_base = wid * ROWS_PER_WORKER

    def idx_src(g):
        return x_hbm.at[pl.ds(row_base + g * G, G)]

    def fire(par, gsem):
        for r in range(G):
            for off, n in ((0, 104), (104, 96)):
                pltpu.async_copy(
                    emb_hbm.at[idxb.at[par, r, pl.ds(off, n)]],
                    gbuf.at[par, pl.ds(r * L + off, n)],
                    gsem,
                )

    def drain(par, gsem):
        pltpu.make_async_copy(
            emb_hbm.at[pl.ds(0, GROW)], gbuf.at[par], gsem
        ).wait()

    def reduce_group(g, par):
        zero = jnp.zeros((16,), jnp.float32)
        for r in range(G):
            ro = r * L

            def red(l2, accs, _ro=ro):
                a0, a1 = accs
                for u in range(UNROLL):
                    off = _ro + l2 * UNROLL + u
                    a0 = a0 + gbuf[par, off, pl.ds(0, 16)]
                    a1 = a1 + gbuf[par, off, pl.ds(16, 16)]
                return (a0, a1)

            a0, a1 = lax.fori_loop(0, L // UNROLL, red, (zero, zero))
            out_row = g * G + r
            sums_v[out_row, pl.ds(0, 16)] = a0
            sums_v[out_row, pl.ds(16, 16)] = a1

    def group_iter(g, par, gsem_cur, gsem_next, isem_next, isem_cur):
        drain(par, gsem_cur)

        @pl.when(g + 1 < NG)
        def _():
            pltpu.make_async_copy(idx_src(g + 1), idxb.at[1 - par],
                                  isem_next).wait()
            fire(1 - par, gsem_next)

        @pl.when(g + 2 < NG)
        def _():
            pltpu.async_copy(idx_src(g + 2), idxb.at[par], isem_cur)

        reduce_group(g, par)

    pltpu.sync_copy(idx_src(0), idxb.at[0])
    fire(0, gsem0)
    pltpu.async_copy(idx_src(1), idxb.at[1], isem1)

    def two_groups(gp, _):
        g = 2 * gp
        group_iter(g, 0, gsem0, gsem1, isem1, isem0)
        group_iter(g + 1, 1, gsem1, gsem0, isem0, isem1)
        return 0

    lax.fori_loop(0, NG // 2, two_groups, 0)
    pltpu.sync_copy(sums_v, out_hbm.at[pl.ds(row_base, ROWS_PER_WORKER)])


def _sc_pooled_sums(x_r, emb):
    mesh = plsc.VectorSubcoreMesh(core_axis_name="c", subcore_axis_name="s")
    return pl.kernel(
        _sc_body,
        out_type=jax.ShapeDtypeStruct((B, EMB_DIM), jnp.float32),
        mesh=mesh,
        compiler_params=pltpu.CompilerParams(use_tc_tiling_on_sc=False),
        scratch_types=[
            pltpu.VMEM((2, G, L), jnp.int32),
            pltpu.VMEM((2, GROW, EMB_DIM), jnp.float32),
            pltpu.VMEM((ROWS_PER_WORKER, EMB_DIM), jnp.float32),
            pltpu.SemaphoreType.DMA,
            pltpu.SemaphoreType.DMA,
            pltpu.SemaphoreType.DMA,
            pltpu.SemaphoreType.DMA,
        ],
    )(x_r, emb)


def _tc_body(x_ref, sums_ref, w_ref, b_ref, out_ref):
    cnt = jnp.sum((x_ref[...] != 0).astype(jnp.float32), axis=1, keepdims=True)
    denom = jnp.maximum(cnt, 1.0)
    avg = sums_ref[...] / denom
    out_ref[...] = (
        lax.dot_general(avg, w_ref[...], (((1,), (1,)), ((), ())),
                        preferred_element_type=jnp.float32)
        + b_ref[...]
    )


def _tc_logits(x, sums, fc_w, fc_b):
    blk = 2048
    grid = (B // blk,)
    return pl.pallas_call(
        _tc_body,
        grid=grid,
        in_specs=[
            pl.BlockSpec((blk, L), lambda i: (i, 0)),
            pl.BlockSpec((blk, EMB_DIM), lambda i: (i, 0)),
            pl.BlockSpec((NUM_CLASSES, EMB_DIM), lambda i: (0, 0)),
            pl.BlockSpec((1, NUM_CLASSES), lambda i: (0, 0)),
        ],
        out_specs=pl.BlockSpec((blk, NUM_CLASSES), lambda i: (i, 0)),
        out_shape=jax.ShapeDtypeStruct((B, NUM_CLASSES), jnp.float32),
    )(x, sums, fc_w, fc_b)


TCHUNK = 2048
NTBLK = (VOCAB + TCHUNK - 1) // TCHUNK
VOCAB_PAD = NTBLK * TCHUNK


def _tr_body(in_ref, out_ref):
    blk = in_ref[...]
    for a in range(4):
        out_ref[:, 32 * a:32 * (a + 1)] = blk[:, 512 * a:512 * (a + 1)].T


def _emb_to_scformat(emb):
    embt = emb.T
    o = pl.pallas_call(
        _tr_body,
        grid=(NTBLK,),
        in_specs=[pl.BlockSpec((EMB_DIM, TCHUNK), lambda i: (0, i))],
        out_specs=pl.BlockSpec((TCHUNK // 4, 128), lambda i: (i, 0)),
        out_shape=jax.ShapeDtypeStruct((VOCAB_PAD // 4, 128), jnp.float32),
    )(embt)
    return o.reshape(-1).reshape(VOCAB_PAD, EMB_DIM)


def _remap_ids(t):
    return (t & ~2047) + ((t & 511) << 2) + ((t & 2047) >> 9)


def kernel(x, emb, fc_w, fc_b):
    x = x.astype(jnp.int32)
    emb2 = _emb_to_scformat(emb)
    x_lin = lax.optimization_barrier(x.reshape(-1))
    x2 = _remap_ids(x_lin).reshape(B, L)
    sums = _sc_pooled_sums(x2, emb2)
    return _tc_logits(x2, sums, fc_w, fc_b.reshape(1, NUM_CLASSES))


# speedup vs baseline: 28.1798x; 1.6027x over previous
import functools

import jax
import jax.numpy as jnp
from jax import lax
from jax.experimental import pallas as pl
from jax.experimental.pallas import tpu as pltpu
from jax.experimental.pallas import tpu_sc as plsc

VOCAB = 1000000
NUM_CLASSES = 100
EMB_DIM = 32
B = 16384
L = 200
HALF_L = L // 2

NUM_WORKERS = 32
ROWS_PER_WORKER = B // NUM_WORKERS


G = 8
NG = ROWS_PER_WORKER // G
GROW = G * L
UNROLL = 8


def _sc_body(x_hbm, emb_hbm, out_hbm, idxb, gbuf, sums_v,
             gsem0, gsem1, isem0, isem1):
    nc = 2
    wid = lax.axis_index("s") * nc + lax.axis_index("c")
    row_base = wid * ROWS_PER_WORKER

    def idx_src(g):
        return x_hbm.at[pl.ds(row_base + g * G, G)]

    def fire(par, gsem):
        for r in range(G):
            for off, n in ((0, 104), (104, 96)):
                pltpu.async_copy(
                    emb_hbm.at[idxb.at[par, r, pl.ds(off, n)]],
                    gbuf.at[par, pl.ds(r * L + off, n)],
                    gsem,
                )

    def drain(par, gsem):
        pltpu.make_async_copy(
            emb_hbm.at[pl.ds(0, GROW)], gbuf.at[par], gsem
        ).wait()

    def reduce_group(g, par):
        zero = jnp.zeros((16,), jnp.float32)
        for r in range(G):
            ro = r * L

            def red(l2, accs, _ro=ro):
                a0, a1 = accs
                for u in range(UNROLL):
                    off = _ro + l2 * UNROLL + u
                    a0 = a0 + gbuf[par, off, pl.ds(0, 16)]
                    a1 = a1 + gbuf[par, off, pl.ds(16, 16)]
                return (a0, a1)

            a0, a1 = lax.fori_loop(0, L // UNROLL, red, (zero, zero))
            out_row = g * G + r
            sums_v[out_row, pl.ds(0, 16)] = a0
            sums_v[out_row, pl.ds(16, 16)] = a1

    def group_iter(g, par, gsem_cur, gsem_next, isem_next, isem_cur):
        drain(par, gsem_cur)

        @pl.when(g + 1 < NG)
        def _():
            pltpu.make_async_copy(idx_src(g + 1), idxb.at[1 - par],
                                  isem_next).wait()
            fire(1 - par, gsem_next)

        @pl.when(g + 2 < NG)
        def _():
            pltpu.async_copy(idx_src(g + 2), idxb.at[par], isem_cur)

        reduce_group(g, par)

    pltpu.sync_copy(idx_src(0), idxb.at[0])
    fire(0, gsem0)
    pltpu.async_copy(idx_src(1), idxb.at[1], isem1)

    def two_groups(gp, _):
        g = 2 * gp
        group_iter(g, 0, gsem0, gsem1, isem1, isem0)
        group_iter(g + 1, 1, gsem1, gsem0, isem0, isem1)
        return 0

    lax.fori_loop(0, NG // 2, two_groups, 0)
    pltpu.sync_copy(sums_v, out_hbm.at[pl.ds(row_base, ROWS_PER_WORKER)])


def _sc_pooled_sums(x_r, emb):
    mesh = plsc.VectorSubcoreMesh(core_axis_name="c", subcore_axis_name="s")
    return pl.kernel(
        _sc_body,
        out_type=jax.ShapeDtypeStruct((B, EMB_DIM), jnp.float32),
        mesh=mesh,
        compiler_params=pltpu.CompilerParams(use_tc_tiling_on_sc=False),
        scratch_types=[
            pltpu.VMEM((2, G, L), jnp.int32),
            pltpu.VMEM((2, GROW, EMB_DIM), jnp.float32),
            pltpu.VMEM((ROWS_PER_WORKER, EMB_DIM), jnp.float32),
            pltpu.SemaphoreType.DMA,
            pltpu.SemaphoreType.DMA,
            pltpu.SemaphoreType.DMA,
            pltpu.SemaphoreType.DMA,
        ],
    )(x_r, emb)


def _tc_body(x_ref, sums_ref, w_ref, b_ref, out_ref):
    cnt = jnp.sum((x_ref[...] != 0).astype(jnp.float32), axis=1, keepdims=True)
    denom = jnp.maximum(cnt, 1.0)
    avg = sums_ref[...] / denom
    out_ref[...] = (
        lax.dot_general(avg, w_ref[...], (((1,), (1,)), ((), ())),
                        preferred_element_type=jnp.float32)
        + b_ref[...]
    )


def _tc_logits(x, sums, fc_w, fc_b):
    blk = 2048
    grid = (B // blk,)
    return pl.pallas_call(
        _tc_body,
        grid=grid,
        in_specs=[
            pl.BlockSpec((blk, L), lambda i: (i, 0)),
            pl.BlockSpec((blk, EMB_DIM), lambda i: (i, 0)),
            pl.BlockSpec((NUM_CLASSES, EMB_DIM), lambda i: (0, 0)),
            pl.BlockSpec((1, NUM_CLASSES), lambda i: (0, 0)),
        ],
        out_specs=pl.BlockSpec((blk, NUM_CLASSES), lambda i: (i, 0)),
        out_shape=jax.ShapeDtypeStruct((B, NUM_CLASSES), jnp.float32),
    )(x, sums, fc_w, fc_b)


TCHUNK = 16384
QCH = TCHUNK // 4
NTBLK = (VOCAB + TCHUNK - 1) // TCHUNK
VOCAB_PAD = NTBLK * TCHUNK


def _tr_body(in_ref, r_ref, out_ref):
    blk = in_ref[...]
    acc = jnp.zeros((QCH, 128), jnp.float32)
    for a in range(4):
        acc = acc + lax.dot_general(
            blk[:, QCH * a:QCH * (a + 1)], r_ref[a],
            (((0,), (0,)), ((), ())), preferred_element_type=jnp.float32)
    out_ref[...] = acc


def _emb_to_scformat(emb):
    embt = emb.T
    d_i = lax.broadcasted_iota(jnp.int32, (4, EMB_DIM, 128), 1)
    c_i = lax.broadcasted_iota(jnp.int32, (4, EMB_DIM, 128), 2)
    a_i = lax.broadcasted_iota(jnp.int32, (4, EMB_DIM, 128), 0)
    r_mats = (c_i == 32 * a_i + d_i).astype(jnp.float32)
    o = pl.pallas_call(
        _tr_body,
        grid=(NTBLK,),
        in_specs=[
            pl.BlockSpec((EMB_DIM, TCHUNK), lambda i: (0, i)),
            pl.BlockSpec((4, EMB_DIM, 128), lambda i: (0, 0, 0)),
        ],
        out_specs=pl.BlockSpec((QCH, 128), lambda i: (i, 0)),
        out_shape=jax.ShapeDtypeStruct((VOCAB_PAD // 4, 128), jnp.float32),
    )(embt, r_mats)
    return o.reshape(-1).reshape(VOCAB_PAD, EMB_DIM)


_QSH = QCH.bit_length() - 1


def _remap_ids(t):
    return (t & ~(TCHUNK - 1)) + ((t & (QCH - 1)) << 2) + ((t & (TCHUNK - 1)) >> _QSH)


def kernel(x, emb, fc_w, fc_b):
    x = x.astype(jnp.int32)
    emb2 = _emb_to_scformat(emb)
    x_lin = lax.optimization_barrier(x.reshape(-1))
    x2 = _remap_ids(x_lin).reshape(B, L)
    sums = _sc_pooled_sums(x2, emb2)
    return _tc_logits(x2, sums, fc_w, fc_b.reshape(1, NUM_CLASSES))


# speedup vs baseline: 30.1483x; 1.0699x over previous
import jax
import jax.numpy as jnp
from jax import lax
from jax.experimental import pallas as pl
from jax.experimental.pallas import tpu as pltpu
from jax.experimental.pallas import tpu_sc as plsc

VOCAB = 1000000
NUM_CLASSES = 100
EMB_DIM = 32
PDIM = EMB_DIM // 2
B = 16384
L = 200

NUM_WORKERS = 32
ROWS_PER_WORKER = B // NUM_WORKERS

G = 16
NG = ROWS_PER_WORKER // G
GROW = G * L
UNROLL = 8

TCHUNK = 16384
QC8 = TCHUNK // 8
NTBLK = (VOCAB + TCHUNK - 1) // TCHUNK
VOCAB_PAD = NTBLK * TCHUNK
_QSH = QC8.bit_length() - 1


def _sc_body(x_hbm, emb_hbm, out_hbm, idxb, gbuf, sums_v,
             gsem0, gsem1, isem0, isem1):
    nc = 2
    wid = lax.axis_index("s") * nc + lax.axis_index("c")
    row_base = wid * ROWS_PER_WORKER

    def idx_src(g):
        return x_hbm.at[pl.ds(row_base + g * G, G)]

    def fire(par, gsem):
        for r in range(G):
            for off, n in ((0, 104), (104, 96)):
                pltpu.async_copy(
                    emb_hbm.at[idxb.at[par, r, pl.ds(off, n)]],
                    gbuf.at[par, pl.ds(r * L + off, n)],
                    gsem,
                )

    def drain(par, gsem):
        pltpu.make_async_copy(
            emb_hbm.at[pl.ds(0, GROW)], gbuf.at[par], gsem
        ).wait()

    def reduce_group(g, par):
        zero = jnp.zeros((16,), jnp.float32)
        himask = jnp.full((16,), -65536, jnp.int32)
        sh64k = jnp.full((16,), 65536, jnp.int32)

        def unpack(off):
            vi = lax.bitcast_convert_type(gbuf[par, off, pl.ds(0, PDIM)], jnp.int32)
            lo = lax.bitcast_convert_type(vi * sh64k, jnp.float32)
            hi = lax.bitcast_convert_type(lax.bitwise_and(vi, himask), jnp.float32)
            return lo, hi

        for r in range(G):
            ro = r * L

            def red(l2, accs, _ro=ro):
                a_lo, a_hi = accs
                for u in range(UNROLL):
                    lo, hi = unpack(_ro + l2 * UNROLL + u)
                    a_lo = a_lo + lo
                    a_hi = a_hi + hi
                return (a_lo, a_hi)

            a_lo, a_hi = lax.fori_loop(0, L // UNROLL, red, (zero, zero))
            out_row = g * G + r
            sums_v[out_row, pl.ds(0, 16)] = a_lo
            sums_v[out_row, pl.ds(16, 16)] = a_hi

    def group_iter(g, par, gsem_cur, gsem_next, isem_next, isem_cur):
        drain(par, gsem_cur)

        @pl.when(g + 1 < NG)
        def _():
            pltpu.make_async_copy(idx_src(g + 1), idxb.at[1 - par],
                                  isem_next).wait()
            fire(1 - par, gsem_next)

        @pl.when(g + 2 < NG)
        def _():
            pltpu.async_copy(idx_src(g + 2), idxb.at[par], isem_cur)

        reduce_group(g, par)

    pltpu.sync_copy(idx_src(0), idxb.at[0])
    fire(0, gsem0)
    pltpu.async_copy(idx_src(1), idxb.at[1], isem1)

    def two_groups(gp, _):
        g = 2 * gp
        group_iter(g, 0, gsem0, gsem1, isem1, isem0)
        group_iter(g + 1, 1, gsem1, gsem0, isem0, isem1)
        return 0

    lax.fori_loop(0, NG // 2, two_groups, 0)
    pltpu.sync_copy(sums_v, out_hbm.at[pl.ds(row_base, ROWS_PER_WORKER)])


def _sc_pooled_sums(x2, emb2):
    mesh = plsc.VectorSubcoreMesh(core_axis_name="c", subcore_axis_name="s")
    return pl.kernel(
        _sc_body,
        out_type=jax.ShapeDtypeStruct((B, EMB_DIM), jnp.float32),
        mesh=mesh,
        compiler_params=pltpu.CompilerParams(use_tc_tiling_on_sc=False),
        scratch_types=[
            pltpu.VMEM((2, G, L), jnp.int32),
            pltpu.VMEM((2, GROW, PDIM), jnp.float32),
            pltpu.VMEM((ROWS_PER_WORKER, EMB_DIM), jnp.float32),
            pltpu.SemaphoreType.DMA,
            pltpu.SemaphoreType.DMA,
            pltpu.SemaphoreType.DMA,
            pltpu.SemaphoreType.DMA,
        ],
    )(x2, emb2)


def _tc_body(x_ref, sums_ref, w_ref, b_ref, out_ref):
    cnt = jnp.sum((x_ref[...] != 0).astype(jnp.float32), axis=1, keepdims=True)
    denom = jnp.maximum(cnt, 1.0)
    avg = sums_ref[...] / denom
    out_ref[...] = (
        lax.dot_general(avg, w_ref[...], (((1,), (1,)), ((), ())),
                        preferred_element_type=jnp.float32)
        + b_ref[...]
    )


def _tc_logits(x, sums, fc_w, fc_b):
    blk = 2048
    return pl.pallas_call(
        _tc_body,
        grid=(B // blk,),
        in_specs=[
            pl.BlockSpec((blk, L), lambda i: (i, 0)),
            pl.BlockSpec((blk, EMB_DIM), lambda i: (i, 0)),
            pl.BlockSpec((NUM_CLASSES, EMB_DIM), lambda i: (0, 0)),
            pl.BlockSpec((1, NUM_CLASSES), lambda i: (0, 0)),
        ],
        out_specs=pl.BlockSpec((blk, NUM_CLASSES), lambda i: (i, 0)),
        out_shape=jax.ShapeDtypeStruct((B, NUM_CLASSES), jnp.float32),
    )(x, sums, fc_w, fc_b)


def _tr_body(in_ref, r_ref, out_ref):
    blk = in_ref[...].astype(jnp.bfloat16)
    lo = jnp.zeros((QC8, 128), jnp.float32)
    hi = jnp.zeros((QC8, 128), jnp.float32)
    for s in range(8):
        sub = blk[:, QC8 * s:QC8 * (s + 1)]
        lo = lo + lax.dot_general(sub, r_ref[s, 0], (((0,), (0,)), ((), ())),
                                  preferred_element_type=jnp.float32)
        hi = hi + lax.dot_general(sub, r_ref[s, 1], (((0,), (0,)), ((), ())),
                                  preferred_element_type=jnp.float32)
    lo_i = lax.bitcast_convert_type(lo, jnp.int32)
    hi_i = lax.bitcast_convert_type(hi, jnp.int32)
    packed = lax.shift_right_logical(lo_i, 16) | (hi_i & jnp.int32(-65536))
    out_ref[...] = lax.bitcast_convert_type(packed, jnp.float32)


def _emb_to_scformat(emb):
    embt = emb.T
    s_i = lax.broadcasted_iota(jnp.int32, (8, 2, EMB_DIM, 128), 0)
    p_i = lax.broadcasted_iota(jnp.int32, (8, 2, EMB_DIM, 128), 1)
    d_i = lax.broadcasted_iota(jnp.int32, (8, 2, EMB_DIM, 128), 2)
    m_i = lax.broadcasted_iota(jnp.int32, (8, 2, EMB_DIM, 128), 3)
    r_mats = ((m_i == 16 * s_i + d_i // 2)
              & (d_i % 2 == p_i)).astype(jnp.bfloat16)
    o = pl.pallas_call(
        _tr_body,
        grid=(NTBLK,),
        in_specs=[
            pl.BlockSpec((EMB_DIM, TCHUNK), lambda i: (0, i)),
            pl.BlockSpec((8, 2, EMB_DIM, 128), lambda i: (0, 0, 0, 0)),
        ],
        out_specs=pl.BlockSpec((QC8, 128), lambda i: (i, 0)),
        out_shape=jax.ShapeDtypeStruct((VOCAB_PAD // 8, 128), jnp.float32),
    )(embt, r_mats)
    return o.reshape(-1).reshape(VOCAB_PAD, PDIM)


def _remap_ids(t):
    return ((t & ~(TCHUNK - 1)) + ((t & (QC8 - 1)) << 3)
            + ((t & (TCHUNK - 1)) >> _QSH))


def kernel(x, emb, fc_w, fc_b):
    x = x.astype(jnp.int32)
    emb2 = _emb_to_scformat(emb)
    x_lin = lax.optimization_barrier(x.reshape(-1))
    x2 = _remap_ids(x_lin).reshape(B, L)
    sums = _sc_pooled_sums(x2, emb2)
    perm = jnp.arange(EMB_DIM).reshape(PDIM, 2).T.reshape(-1)
    return _tc_logits(x2, sums, fc_w[:, perm], fc_b.reshape(1, NUM_CLASSES))


# speedup vs baseline: 30.3111x; 1.0054x over previous
import jax
import jax.numpy as jnp
from jax import lax
from jax.experimental import pallas as pl
from jax.experimental.pallas import tpu as pltpu
from jax.experimental.pallas import tpu_sc as plsc

VOCAB = 1000000
NUM_CLASSES = 100
EMB_DIM = 32
PDIM = EMB_DIM // 2
B = 16384
L = 200

NUM_WORKERS = 32
ROWS_PER_WORKER = B // NUM_WORKERS

G = 16
NG = ROWS_PER_WORKER // G
GROW = G * L
UNROLL = 8

TCHUNK = 16384
QC8 = TCHUNK // 8
NTBLK = (VOCAB + TCHUNK - 1) // TCHUNK
VOCAB_PAD = NTBLK * TCHUNK
_QSH = QC8.bit_length() - 1


def _sc_body(x_hbm, emb_hbm, out_hbm, idxb, gbuf, sums_v,
             gsem0, gsem1, isem0, isem1):
    nc = 2
    wid = lax.axis_index("s") * nc + lax.axis_index("c")
    row_base = wid * ROWS_PER_WORKER

    def idx_src(g):
        return x_hbm.at[pl.ds(row_base + g * G, G)]

    def fire(par, gsem):
        for r in range(G):
            for off, n in ((0, 104), (104, 96)):
                pltpu.async_copy(
                    emb_hbm.at[idxb.at[par, r, pl.ds(off, n)]],
                    gbuf.at[par, pl.ds(r * L + off, n)],
                    gsem,
                )

    def drain(par, gsem):
        pltpu.make_async_copy(
            emb_hbm.at[pl.ds(0, GROW)], gbuf.at[par], gsem
        ).wait()

    def reduce_group(g, par):
        zero = jnp.zeros((16,), jnp.float32)
        himask = jnp.full((16,), -65536, jnp.int32)
        sh64k = jnp.full((16,), 65536, jnp.int32)

        def unpack(off):
            vi = lax.bitcast_convert_type(gbuf[par, off, pl.ds(0, PDIM)],
                                          jnp.int32)
            lo = lax.bitcast_convert_type(vi * sh64k, jnp.float32)
            hi = lax.bitcast_convert_type(lax.bitwise_and(vi, himask),
                                          jnp.float32)
            return lo, hi

        nacc = 4

        for r in range(G):
            ro = r * L

            def red(l2, accs, _ro=ro):
                accs = list(accs)
                for u in range(UNROLL):
                    lo, hi = unpack(_ro + l2 * UNROLL + u)
                    k = u % nacc
                    accs[2 * k] = accs[2 * k] + lo
                    accs[2 * k + 1] = accs[2 * k + 1] + hi
                return tuple(accs)

            accs = lax.fori_loop(0, L // UNROLL, red, (zero,) * (2 * nacc))
            a_lo = (accs[0] + accs[2]) + (accs[4] + accs[6])
            a_hi = (accs[1] + accs[3]) + (accs[5] + accs[7])
            out_row = g * G + r
            sums_v[out_row, pl.ds(0, 16)] = a_lo
            sums_v[out_row, pl.ds(16, 16)] = a_hi

    def group_iter(g, par, gsem_cur, gsem_next, isem_next, isem_cur):
        drain(par, gsem_cur)

        @pl.when(g + 1 < NG)
        def _():
            pltpu.make_async_copy(idx_src(g + 1), idxb.at[1 - par],
                                  isem_next).wait()
            fire(1 - par, gsem_next)

        @pl.when(g + 2 < NG)
        def _():
            pltpu.async_copy(idx_src(g + 2), idxb.at[par], isem_cur)

        reduce_group(g, par)

    pltpu.sync_copy(idx_src(0), idxb.at[0])
    fire(0, gsem0)
    pltpu.async_copy(idx_src(1), idxb.at[1], isem1)

    def two_groups(gp, _):
        g = 2 * gp
        group_iter(g, 0, gsem0, gsem1, isem1, isem0)
        group_iter(g + 1, 1, gsem1, gsem0, isem0, isem1)
        return 0

    lax.fori_loop(0, NG // 2, two_groups, 0)
    pltpu.sync_copy(sums_v, out_hbm.at[pl.ds(row_base, ROWS_PER_WORKER)])


def _sc_pooled_sums(x2, emb2):
    mesh = plsc.VectorSubcoreMesh(core_axis_name="c", subcore_axis_name="s")
    return pl.kernel(
        _sc_body,
        out_type=jax.ShapeDtypeStruct((B, EMB_DIM), jnp.float32),
        mesh=mesh,
        compiler_params=pltpu.CompilerParams(use_tc_tiling_on_sc=False),
        scratch_types=[
            pltpu.VMEM((2, G, L), jnp.int32),
            pltpu.VMEM((2, GROW, PDIM), jnp.float32),
            pltpu.VMEM((ROWS_PER_WORKER, EMB_DIM), jnp.float32),
            pltpu.SemaphoreType.DMA,
            pltpu.SemaphoreType.DMA,
            pltpu.SemaphoreType.DMA,
            pltpu.SemaphoreType.DMA,
        ],
    )(x2, emb2)


def _tc_body(x_ref, sums_ref, w_ref, b_ref, out_ref):
    cnt = jnp.sum((x_ref[...] != 0).astype(jnp.float32), axis=1, keepdims=True)
    denom = jnp.maximum(cnt, 1.0)
    avg = sums_ref[...] / denom
    out_ref[...] = (
        lax.dot_general(avg, w_ref[...], (((1,), (1,)), ((), ())),
                        preferred_element_type=jnp.float32)
        + b_ref[...]
    )


def _tc_logits(x, sums, fc_w, fc_b):
    blk = 2048
    return pl.pallas_call(
        _tc_body,
        grid=(B // blk,),
        in_specs=[
            pl.BlockSpec((blk, L), lambda i: (i, 0)),
            pl.BlockSpec((blk, EMB_DIM), lambda i: (i, 0)),
            pl.BlockSpec((NUM_CLASSES, EMB_DIM), lambda i: (0, 0)),
            pl.BlockSpec((1, NUM_CLASSES), lambda i: (0, 0)),
        ],
        out_specs=pl.BlockSpec((blk, NUM_CLASSES), lambda i: (i, 0)),
        out_shape=jax.ShapeDtypeStruct((B, NUM_CLASSES), jnp.float32),
    )(x, sums, fc_w, fc_b)


def _tr_body(in_ref, r_ref, out_ref):
    blk = in_ref[...].astype(jnp.bfloat16)
    acc = jnp.zeros((QC8, 256), jnp.float32)
    for s in range(8):
        sub = blk[:, QC8 * s:QC8 * (s + 1)]
        acc = acc + lax.dot_general(sub, r_ref[s],
                                    (((0,), (0,)), ((), ())),
                                    preferred_element_type=jnp.float32)
    lo_i = lax.bitcast_convert_type(acc[:, :128], jnp.int32)
    hi_i = lax.bitcast_convert_type(acc[:, 128:], jnp.int32)
    packed = lax.shift_right_logical(lo_i, 16) | (hi_i & jnp.int32(-65536))
    out_ref[...] = lax.bitcast_convert_type(packed, jnp.float32)


def _emb_to_scformat(emb):
    embt = emb.T
    s_i = lax.broadcasted_iota(jnp.int32, (8, EMB_DIM, 256), 0)
    d_i = lax.broadcasted_iota(jnp.int32, (8, EMB_DIM, 256), 1)
    m_i = lax.broadcasted_iota(jnp.int32, (8, EMB_DIM, 256), 2)
    r_mats = ((m_i % 128 == 16 * s_i + d_i // 2)
              & (d_i % 2 == m_i // 128)).astype(jnp.bfloat16)
    o = pl.pallas_call(
        _tr_body,
        grid=(NTBLK,),
        in_specs=[
            pl.BlockSpec((EMB_DIM, TCHUNK), lambda i: (0, i)),
            pl.BlockSpec((8, EMB_DIM, 256), lambda i: (0, 0, 0)),
        ],
        out_specs=pl.BlockSpec((QC8, 128), lambda i: (i, 0)),
        out_shape=jax.ShapeDtypeStruct((VOCAB_PAD // 8, 128), jnp.float32),
    )(embt, r_mats)
    return o.reshape(-1).reshape(VOCAB_PAD, PDIM)


def _remap_ids(t):
    return ((t & ~(TCHUNK - 1)) + ((t & (QC8 - 1)) << 3)
            + ((t & (TCHUNK - 1)) >> _QSH))


def kernel(x, emb, fc_w, fc_b):
    x = x.astype(jnp.int32)
    emb2 = _emb_to_scformat(emb)
    x_lin = lax.optimization_barrier(x.reshape(-1))
    x2 = _remap_ids(x_lin).reshape(B, L)
    sums = _sc_pooled_sums(x2, emb2)
    perm = jnp.arange(EMB_DIM).reshape(PDIM, 2).T.reshape(-1)
    return _tc_logits(x2, sums, fc_w[:, perm], fc_b.reshape(1, NUM_CLASSES))
